# Initial kernel scaffold; baseline (speedup 1.0000x reference)
import jax
import jax.numpy as jnp
from jax.experimental import pallas as pl


def kernel(x, edge_index, edge_attr, params):
    raise NotImplementedError("write your pallas kernel here")


# speedup vs baseline: 1.5478x; 1.5478x over previous
import functools

import jax
import jax.numpy as jnp
from jax import lax
from jax.experimental import pallas as pl
from jax.experimental.pallas import tpu as pltpu
from jax.experimental.pallas import tpu_sc as plsc

_N = 10000
_E = 320000
_IN = 128
_HID = 64
_RBFK = 16
_NCLS = 26
_WY = 100

_NC = 2
_NS = 16
_NW = _NC * _NS
_CH = 128
_CHP = _CH // 2
_NCHK = 80
_EPW = _NCHK * _CH
_EPAD = _NW * _EPW
_EP2 = _EPAD // 2
_NACC = 10240

_F32 = jnp.float32
_I32 = jnp.int32

_SC_PARAMS = pltpu.CompilerParams(use_tc_tiling_on_sc=False)


def _silu(x):
    return x * jax.nn.sigmoid(x)


def _lnorm(x, g, b):
    mu = jnp.mean(x, axis=-1, keepdims=True)
    var = jnp.mean((x - mu) ** 2, axis=-1, keepdims=True)
    return (x - mu) / jnp.sqrt(var + 1e-5) * g + b


def _dot(a, b):
    return jnp.dot(a, b, preferred_element_type=_F32)


def _rmax_body(vx_ref, vy_ref, vz_ref, out_ref):
    r2 = vx_ref[...] ** 2 + vy_ref[...] ** 2 + vz_ref[...] ** 2
    r = jnp.maximum(jnp.sqrt(r2), 1e-8)
    rb = lax.bitcast_convert_type(r, _I32)
    k1 = _I32(304000)
    k2 = _I32(304001)

    def body(i, c):
        lo1, hi1, lo2, hi2 = c
        mid1 = lo1 + (hi1 - lo1) // 2
        mid2 = lo2 + (hi2 - lo2) // 2
        c1 = jnp.sum((rb <= mid1).astype(_I32))
        c2 = jnp.sum((rb <= mid2).astype(_I32))
        lo1n = jnp.where(c1 >= k1, lo1, mid1)
        hi1n = jnp.where(c1 >= k1, mid1, hi1)
        lo2n = jnp.where(c2 >= k2, lo2, mid2)
        hi2n = jnp.where(c2 >= k2, mid2, hi2)
        return lo1n, hi1n, lo2n, hi2n

    init = (_I32(-1), _I32(0x7F7FFFFF), _I32(-1), _I32(0x7F7FFFFF))
    _, hi1, _, hi2 = lax.fori_loop(0, 31, body, init)
    a = lax.bitcast_convert_type(hi1, _F32)
    b = lax.bitcast_convert_type(hi2, _F32)
    idxf = _F32(0.95) * _F32(_E - 1)
    frac = idxf - jnp.floor(idxf)
    q = a + (b - a) * frac
    out_ref[0, 0] = jnp.clip(q, 1.0, 8.0)


def _rmax_call(vx, vy, vz):
    return pl.pallas_call(
        _rmax_body,
        out_shape=jax.ShapeDtypeStruct((1, 1), _F32),
        out_specs=pl.BlockSpec(memory_space=pltpu.SMEM),
    )(vx, vy, vz)


def _node_body(x_ref, win_ref, bin_ref, g_ref, b_ref, w1ab_ref,
               h_ref, gid_ref, t_ref):
    x = x_ref[...]
    t = _dot(x, win_ref[...]) + bin_ref[...]
    h = _silu(_lnorm(t, g_ref[...], b_ref[...]))
    h_ref[...] = h
    t_ref[...] = _dot(h, w1ab_ref[...])
    xa = x[:, _WY:_WY + _NCLS]
    m = jnp.max(xa, axis=1, keepdims=True)
    io = lax.broadcasted_iota(_I32, (1, _NCLS), 1)
    cand = jnp.where(xa == m, io, _I32(2 ** 30))
    gid_ref[...] = jnp.min(cand, axis=1, keepdims=True)


def _node_call(x, win, bin_, g, b, w1ab):
    bn = 1000
    nb = _N // bn
    wspec = lambda shape: pl.BlockSpec(shape, lambda i: (0, 0))
    return pl.pallas_call(
        _node_body,
        grid=(nb,),
        in_specs=[
            pl.BlockSpec((bn, _IN), lambda i: (i, 0)),
            wspec((_IN, _HID)), wspec((1, _HID)), wspec((1, _HID)),
            wspec((1, _HID)), wspec((_HID, 2 * _HID)),
        ],
        out_specs=[
            pl.BlockSpec((bn, _HID), lambda i: (i, 0)),
            pl.BlockSpec((bn, 1), lambda i: (i, 0)),
            pl.BlockSpec((bn, 2 * _HID), lambda i: (i, 0)),
        ],
        out_shape=[
            jax.ShapeDtypeStruct((_N, _HID), _F32),
            jax.ShapeDtypeStruct((_N, 1), _I32),
            jax.ShapeDtypeStruct((_N, 2 * _HID), _F32),
        ],
    )(x, win, bin_, g, b, w1ab)


def _edge_body(rmax_ref, ea_ref, weblk_ref, beblk_ref, g_ref, b_ref, o_ref):
    ea = ea_ref[...]
    rm = rmax_ref[0, 0]
    io = lax.broadcasted_iota(_I32, (1, _RBFK), 1).astype(_F32)
    centers = rm * io / _F32(_RBFK - 1)
    delta = jnp.maximum(rm / _F32(_RBFK - 1), 1e-3)
    gamma = 1.0 / (2.0 * (0.5 * delta) ** 2)

    def feats(v):
        r = jnp.maximum(jnp.sqrt(jnp.sum(v * v, axis=1, keepdims=True)),
                        1e-8)
        u = v / r
        rbf = jnp.exp(-gamma * (r - centers) ** 2)
        return jnp.concatenate([u, r, rbf], axis=1)

    e1 = feats(ea[:, 0:3])
    e2 = feats(ea[:, 4:7])
    ep = jnp.concatenate([e1, e2], axis=1)
    t = _dot(ep, weblk_ref[...]) + beblk_ref[...]
    s = _silu(t)
    g = g_ref[...]
    b = b_ref[...]
    h1 = _lnorm(s[:, :_HID], g, b)
    h2 = _lnorm(s[:, _HID:], g, b)
    o_ref[...] = jnp.concatenate([h1, h2], axis=1)


def _edge_call(rmax, eap, weblk, beblk, g, b):
    bp = 1024
    nb = _EP2 // bp
    wspec = lambda shape: pl.BlockSpec(shape, lambda i: (0, 0))
    return pl.pallas_call(
        _edge_body,
        grid=(nb,),
        in_specs=[
            pl.BlockSpec(memory_space=pltpu.SMEM),
            pl.BlockSpec((bp, 8), lambda i: (i, 0)),
            wspec((40, 2 * _HID)), wspec((1, 2 * _HID)),
            wspec((1, _HID)), wspec((1, _HID)),
        ],
        out_specs=pl.BlockSpec((bp, 2 * _HID), lambda i: (i, 0)),
        out_shape=jax.ShapeDtypeStruct((_EP2, 2 * _HID), _F32),
    )(rmax, eap, weblk, beblk, g, b)


def _msg_body(s_ref, he_ref, w1c_ref, b1_ref, w2_ref, b2_ref, o_ref):
    m1 = s_ref[...] + _dot(he_ref[...], w1c_ref[...]) + b1_ref[...]
    m1 = _silu(m1)
    o_ref[...] = _silu(_dot(m1, w2_ref[...]) + b2_ref[...])


def _msg_call(s, he, w1cblk, b1blk, w2blk, b2blk):
    bp = 1024
    nb = _EP2 // bp
    espec = pl.BlockSpec((bp, 2 * _HID), lambda i: (i, 0))
    wspec = lambda shape: pl.BlockSpec(shape, lambda i: (0, 0))
    return pl.pallas_call(
        _msg_body,
        grid=(nb,),
        in_specs=[espec, espec,
                  wspec((2 * _HID, 2 * _HID)), wspec((1, 2 * _HID)),
                  wspec((2 * _HID, 2 * _HID)), wspec((1, 2 * _HID))],
        out_specs=espec,
        out_shape=jax.ShapeDtypeStruct((_EP2, 2 * _HID), _F32),
    )(s, he, w1cblk, b1blk, w2blk, b2blk)


def _upd_body_proj(h_ref, m0_ref, m1_ref, wh_ref, wm_ref, bu_ref, g_ref,
                   b_ref, w1ab_ref, h_out, t_out):
    h = h_ref[...]
    m = m0_ref[0] + m1_ref[0]
    t = _silu(_dot(h, wh_ref[...]) + _dot(m, wm_ref[...]) + bu_ref[...]) + h
    hn = _lnorm(t, g_ref[...], b_ref[...])
    h_out[...] = hn
    t_out[...] = _dot(hn, w1ab_ref[...])


def _upd_body(h_ref, m0_ref, m1_ref, wh_ref, wm_ref, bu_ref, g_ref, b_ref,
              h_out):
    h = h_ref[...]
    m = m0_ref[0] + m1_ref[0]
    t = _silu(_dot(h, wh_ref[...]) + _dot(m, wm_ref[...]) + bu_ref[...]) + h
    h_out[...] = _lnorm(t, g_ref[...], b_ref[...])


def _upd_call(h, macc, wh, wm, bu, g, b, w1ab=None):
    bn = 1000
    nb = _N // bn
    nspec = pl.BlockSpec((bn, _HID), lambda i: (i, 0))
    m0spec = pl.BlockSpec((1, bn, _HID), lambda i: (0, i, 0))
    m1spec = pl.BlockSpec((1, bn, _HID), lambda i: (1, i, 0))
    wspec = lambda shape: pl.BlockSpec(shape, lambda i: (0, 0))
    hspec = jax.ShapeDtypeStruct((_N, _HID), _F32)
    if w1ab is None:
        return pl.pallas_call(
            _upd_body,
            grid=(nb,),
            in_specs=[nspec, m0spec, m1spec,
                      wspec((_HID, _HID)), wspec((_HID, _HID)),
                      wspec((1, _HID)), wspec((1, _HID)), wspec((1, _HID))],
            out_specs=nspec,
            out_shape=hspec,
        )(h, macc, macc, wh, wm, bu, g, b)
    return pl.pallas_call(
        _upd_body_proj,
        grid=(nb,),
        in_specs=[nspec, m0spec, m1spec,
                  wspec((_HID, _HID)), wspec((_HID, _HID)),
                  wspec((1, _HID)), wspec((1, _HID)), wspec((1, _HID)),
                  wspec((_HID, 2 * _HID))],
        out_specs=[nspec, pl.BlockSpec((bn, 2 * _HID), lambda i: (i, 0))],
        out_shape=[hspec, jax.ShapeDtypeStruct((_N, 2 * _HID), _F32)],
    )(h, macc, macc, wh, wm, bu, g, b, w1ab)


def _f1_body(s0_ref, s1_ref, s2_ref, s3_ref, gid_ref, wj0_ref, wj1_ref,
             wj2_ref, wj3_ref, bj_ref, wo1_ref, bo1_ref, wo2_ref, bo2_ref,
             sums_ref, cnts_ref):
    hagg = (_dot(s0_ref[...], wj0_ref[...]) + _dot(s1_ref[...], wj1_ref[...])
            + _dot(s2_ref[...], wj2_ref[...]) + _dot(s3_ref[...], wj3_ref[...])
            + bj_ref[...])
    z = _silu(_dot(hagg, wo1_ref[...]) + bo1_ref[...])
    lg = _dot(z, wo2_ref[...]) + bo2_ref[...]
    grow = gid_ref[0]
    io = lax.broadcasted_iota(_I32, (_NCLS, 1), 0)
    oh = (grow == io).astype(_F32)

    @pl.when(pl.program_id(0) == 0)
    def _():
        sums_ref[...] = jnp.zeros_like(sums_ref)
        cnts_ref[...] = jnp.zeros_like(cnts_ref)

    sums_ref[...] += _dot(oh, lg)
    cnts_ref[...] += jnp.sum(oh, axis=1, keepdims=True)


def _f1_call(s0, s1, s2, s3, gid3, wj0, wj1, wj2, wj3, bj, wo1, bo1, wo2,
             bo2):
    bn = 1000
    nb = _N // bn
    nspec = pl.BlockSpec((bn, _HID), lambda i: (i, 0))
    wspec = lambda shape: pl.BlockSpec(shape, lambda i: (0, 0))
    return pl.pallas_call(
        _f1_body,
        grid=(nb,),
        in_specs=[nspec, nspec, nspec, nspec,
                  pl.BlockSpec((1, 1, bn), lambda i: (i, 0, 0)),
                  wspec((_HID, _HID)), wspec((_HID, _HID)),
                  wspec((_HID, _HID)), wspec((_HID, _HID)),
                  wspec((1, _HID)), wspec((_HID, _HID)), wspec((1, _HID)),
                  wspec((_HID, _NCLS)), wspec((1, _NCLS))],
        out_specs=[pl.BlockSpec((_NCLS, _NCLS), lambda i: (0, 0)),
                   pl.BlockSpec((_NCLS, 1), lambda i: (0, 0))],
        out_shape=[jax.ShapeDtypeStruct((_NCLS, _NCLS), _F32),
                   jax.ShapeDtypeStruct((_NCLS, 1), _F32)],
        compiler_params=pltpu.CompilerParams(
            dimension_semantics=("arbitrary",)),
    )(s0, s1, s2, s3, gid3, wj0, wj1, wj2, wj3, bj, wo1, bo1, wo2, bo2)


def _f2_body(gid_ref, sums_ref, cnts_ref, o_ref):
    mean = sums_ref[...] / jnp.maximum(cnts_ref[...], 1.0)
    gid = gid_ref[...]
    io = lax.broadcasted_iota(_I32, (1, _NCLS), 1)
    oh = (gid == io).astype(_F32)
    o_ref[...] = _dot(oh, mean)


def _f2_call(gid, sums, cnts):
    bn = 1000
    nb = _N // bn
    return pl.pallas_call(
        _f2_body,
        grid=(nb,),
        in_specs=[pl.BlockSpec((bn, 1), lambda i: (i, 0)),
                  pl.BlockSpec((_NCLS, _NCLS), lambda i: (0, 0)),
                  pl.BlockSpec((_NCLS, 1), lambda i: (0, 0))],
        out_specs=pl.BlockSpec((bn, _NCLS), lambda i: (i, 0)),
        out_shape=jax.ShapeDtypeStruct((_N, _NCLS), _F32),
    )(gid, sums, cnts)


def _sc_gather(tbl, src2d, dst2d):
    mesh = plsc.VectorSubcoreMesh(core_axis_name="c", subcore_axis_name="s")

    @functools.partial(
        pl.kernel, mesh=mesh,
        out_type=jax.ShapeDtypeStruct((_EP2, 2 * _HID), _F32),
        scratch_types=[
            pltpu.VMEM((_NCHK, _CH), _I32),
            pltpu.VMEM((_NCHK, _CH), _I32),
            pltpu.VMEM((_CH, 2 * _HID), _F32),
            pltpu.VMEM((_CH, 2 * _HID), _F32),
            pltpu.VMEM((_CH, 2 * _HID), _F32),
            pltpu.VMEM((_CH, 2 * _HID), _F32),
            pltpu.VMEM((_CHP, 2 * _HID), _F32),
            pltpu.VMEM((_CHP, 2 * _HID), _F32),
            pltpu.SemaphoreType.DMA,
            pltpu.SemaphoreType.DMA,
            pltpu.SemaphoreType.DMA,
            pltpu.SemaphoreType.DMA,
        ],
        compiler_params=_SC_PARAMS,
    )
    def k(tbl_hbm, src_hbm, dst_hbm, s_hbm,
          idx1, idx2, bu0, bv0, bu1, bv1, bs0, bs1, gs0, gs1, ws0, ws1):
        wid = lax.axis_index("s") * _NC + lax.axis_index("c")
        rowbase = wid * _NCHK
        pltpu.sync_copy(src_hbm.at[pl.ds(rowbase, _NCHK)], idx1)
        pltpu.sync_copy(dst_hbm.at[pl.ds(rowbase, _NCHK)], idx2)
        pbase = wid * (_EPW // 2)

        def combine(bu, bv, bs):
            def cb(kk, carry):
                for half in range(2):
                    r = 2 * kk + half
                    for c in range(4):
                        lo = half * _HID + c * 16
                        bs[kk, pl.ds(lo, 16)] = (
                            bu[r, pl.ds(c * 16, 16)]
                            + bv[r, pl.ds(_HID + c * 16, 16)])
                return carry

            lax.fori_loop(0, _CHP, cb, 0)

        def drain(sem, bs):
            pltpu.make_async_copy(bs, s_hbm.at[pl.ds(0, _CHP)], sem).wait()

        def body(i, carry):
            e = 2 * i
            o = 2 * i + 1

            @pl.when(i > 0)
            def _():
                drain(ws0, bs0)
                drain(ws1, bs1)

            c1 = pltpu.async_copy(tbl_hbm.at[idx1.at[e]], bu0, gs0)
            c2 = pltpu.async_copy(tbl_hbm.at[idx2.at[e]], bv0, gs0)
            c3 = pltpu.async_copy(tbl_hbm.at[idx1.at[o]], bu1, gs1)
            c4 = pltpu.async_copy(tbl_hbm.at[idx2.at[o]], bv1, gs1)
            c1.wait()
            c2.wait()
            combine(bu0, bv0, bs0)
            pltpu.async_copy(bs0, s_hbm.at[pl.ds(pbase + e * _CHP, _CHP)],
                             ws0)
            c3.wait()
            c4.wait()
            combine(bu1, bv1, bs1)
            pltpu.async_copy(bs1, s_hbm.at[pl.ds(pbase + o * _CHP, _CHP)],
                             ws1)
            return carry

        lax.fori_loop(0, _NCHK // 2, body, 0)
        drain(ws0, bs0)
        drain(ws1, bs1)

    return k(tbl, src2d, dst2d)


def _sc_scatter(m2, dst2d, zeros_acc):
    mesh = plsc.VectorSubcoreMesh(core_axis_name="c", subcore_axis_name="s")

    @functools.partial(
        pl.kernel, mesh=mesh,
        out_type=jax.ShapeDtypeStruct((_NC, _NACC, _HID), _F32),
        scratch_types=[
            pltpu.VMEM((_NCHK, _CH), _I32),
            pltpu.VMEM((_CHP, 2 * _HID), _F32),
            pltpu.VMEM((_CHP, 2 * _HID), _F32),
            pltpu.VMEM((_CH, _HID), _F32),
            pltpu.VMEM((_CH, _HID), _F32),
            pltpu.VMEM_SHARED((_NACC, _HID), _F32),
            pltpu.SemaphoreType.DMA,
            pltpu.SemaphoreType.DMA,
        ],
        compiler_params=_SC_PARAMS,
    )
    def k(m2_hbm, dst_hbm, zero_hbm, out_hbm, idx, bp0, bp1, bu0, bu1,
          shacc, rs0, rs1):
        c = lax.axis_index("c")
        s = lax.axis_index("s")
        wid = s * _NC + c
        zrows = _NACC // _NS
        pltpu.sync_copy(zero_hbm.at[pl.ds(s * zrows, zrows)],
                        shacc.at[pl.ds(s * zrows, zrows)])
        plsc.subcore_barrier()
        pltpu.sync_copy(dst_hbm.at[pl.ds(wid * _NCHK, _NCHK)], idx)
        pbase = wid * (_EPW // 2)

        def unpack(bp, bu):
            def cb(kk, carry):
                for half in range(2):
                    r = 2 * kk + half
                    for cc in range(4):
                        bu[r, pl.ds(cc * 16, 16)] = bp[
                            kk, pl.ds(half * _HID + cc * 16, 16)]
                return carry

            lax.fori_loop(0, _CHP, cb, 0)

        pltpu.async_copy(m2_hbm.at[pl.ds(pbase, _CHP)], bp0, rs0)

        def body(i, carry):
            e = 2 * i
            o = 2 * i + 1
            pltpu.async_copy(
                m2_hbm.at[pl.ds(pbase + o * _CHP, _CHP)], bp1, rs1)
            pltpu.make_async_copy(
                m2_hbm.at[pl.ds(0, _CHP)], bp0, rs0).wait()
            unpack(bp0, bu0)
            pltpu.sync_copy(bu0, shacc.at[idx.at[e]], add=True)

            @pl.when(i + 1 < _NCHK // 2)
            def _():
                pltpu.async_copy(
                    m2_hbm.at[pl.ds(pbase + (e + 2) * _CHP, _CHP)], bp0, rs0)

            pltpu.make_async_copy(
                m2_hbm.at[pl.ds(0, _CHP)], bp1, rs1).wait()
            unpack(bp1, bu1)
            pltpu.sync_copy(bu1, shacc.at[idx.at[o]], add=True)
            return carry

        lax.fori_loop(0, _NCHK // 2, body, 0)
        plsc.subcore_barrier()
        orows = _NACC // _NS
        pltpu.sync_copy(shacc.at[pl.ds(s * orows, orows)],
                        out_hbm.at[c, pl.ds(s * orows, orows)])

    return k(m2, dst2d, zeros_acc)


def _blkdiag(w):
    z = jnp.zeros_like(w)
    return jnp.concatenate(
        [jnp.concatenate([w, z], axis=1),
         jnp.concatenate([z, w], axis=1)], axis=0)


def kernel(x, edge_index, edge_attr, params):
    p = params
    src = edge_index[0]
    dst = edge_index[1]
    npad = _EPAD - _E
    src2d = jnp.concatenate(
        [src, jnp.zeros((npad,), _I32)]).reshape(_EPAD // _CH, _CH)
    dst2d = jnp.concatenate(
        [dst, jnp.zeros((npad,), _I32)]).reshape(_EPAD // _CH, _CH)
    dsts2d = jnp.concatenate(
        [dst, jnp.full((npad,), _N, _I32)]).reshape(_EPAD // _CH, _CH)
    eap = jnp.concatenate(
        [edge_attr, jnp.zeros((npad, 4), _F32)], axis=0).reshape(_EP2, 8)
    vx = edge_attr[:, 0].reshape(_E // _CH, _CH)
    vy = edge_attr[:, 1].reshape(_E // _CH, _CH)
    vz = edge_attr[:, 2].reshape(_E // _CH, _CH)
    zeros_acc = jnp.zeros((_NACC, _HID), _F32)

    row = lambda v: v.reshape(1, -1)
    two = lambda v: jnp.concatenate([v, v]).reshape(1, -1)
    w1ab = [jnp.concatenate([p['msg_W1'][l][:_HID],
                             p['msg_W1'][l][_HID:2 * _HID]], axis=1)
            for l in range(3)]
    w1cblk = [_blkdiag(p['msg_W1'][l][2 * _HID:]) for l in range(3)]
    w2blk = [_blkdiag(p['msg_W2'][l]) for l in range(3)]
    wuh = [p['upd_W'][l][:_HID] for l in range(3)]
    wum = [p['upd_W'][l][_HID:] for l in range(3)]
    weblk = _blkdiag(p['W_e'])

    rmax = _rmax_call(vx, vy, vz)
    h, gid, tbl = _node_call(
        x, p['W_in'], row(p['b_in']), row(p['ln_in_g']), row(p['ln_in_b']),
        w1ab[0])
    he = _edge_call(rmax, eap, weblk, two(p['b_e']),
                    row(p['ln_e_g']), row(p['ln_e_b']))

    states = [h]
    for l in range(3):
        sg = _sc_gather(tbl, src2d, dst2d)
        m2 = _msg_call(sg, he, w1cblk[l], two(p['msg_b1'][l]),
                       w2blk[l], two(p['msg_b2'][l]))
        macc = _sc_scatter(m2, dsts2d, zeros_acc)
        if l < 2:
            h, tbl = _upd_call(
                h, macc, wuh[l], wum[l], row(p['upd_b'][l]),
                row(p['ln_g'][l]), row(p['ln_b'][l]), w1ab[l + 1])
        else:
            h = _upd_call(
                h, macc, wuh[l], wum[l], row(p['upd_b'][l]),
                row(p['ln_g'][l]), row(p['ln_b'][l]))
        states.append(h)

    wjk = [p['W_jk'][l * _HID:(l + 1) * _HID] for l in range(4)]
    gid3 = gid.reshape(_N // 1000, 1, 1000)
    sums, cnts = _f1_call(states[0], states[1], states[2], states[3], gid3,
                          wjk[0], wjk[1], wjk[2], wjk[3], row(p['b_jk']),
                          p['W_o1'], row(p['b_o1']), p['W_o2'],
                          row(p['b_o2']))
    return _f2_call(gid, sums, cnts)
